# Initial kernel scaffold; baseline (speedup 1.0000x reference)
import jax
import jax.numpy as jnp
from jax.experimental import pallas as pl


def kernel(node_features, timestamps, edge_features, edge_index, W_msg, b_msg, W_qkv, b_qkv, W_o, b_o, W_fc, b_fc):
    raise NotImplementedError("write your pallas kernel here")


# speedup vs baseline: 1.0287x; 1.0287x over previous
import functools
import math

import jax
import jax.numpy as jnp
from jax.experimental import pallas as pl

B = 64
NUM_NODES = 325
E = 940
NODE_DIM = 2
EDGE_DIM = 2
TIME_DIM = 8
OUT = 64
HEADS = 2
D_H = OUT // HEADS

N_P = 384
E_P = 1024


def _fused_kernel(node_ref, ts_ref, ef_ref, src_ref, dst_ref,
                  w12_ref, w34_ref, bmsg_ref, wqkv_ref, bqkv_ref,
                  wo_ref, bo_ref, wfct_ref, bfc_ref, freqs_ref,
                  out_ref):
    f32 = jnp.float32
    node = node_ref[0]
    p12 = jnp.dot(node, w12_ref[0:2, :], preferred_element_type=f32)
    n_iota = jax.lax.broadcasted_iota(jnp.int32, (E_P, N_P), 1)
    oh_src = (n_iota == src_ref[...]).astype(f32)
    oh_dst = (n_iota == dst_ref[...]).astype(f32)
    h = jnp.dot(oh_src, p12[:, :OUT], preferred_element_type=f32)
    h = h + jnp.dot(oh_dst, p12[:, OUT:], preferred_element_type=f32)

    t = ts_ref[0]
    ang = t * freqs_ref[...]
    edte = jnp.concatenate(
        [ef_ref[0], jnp.sin(ang), jnp.cos(ang),
         jnp.zeros((E_P, 6), dtype=f32)], axis=1)
    h = h + jnp.dot(edte, w34_ref[...], preferred_element_type=f32)
    h = h + bmsg_ref[...]

    qkv = jnp.dot(h, wqkv_ref[...], preferred_element_type=f32) + bqkv_ref[...]
    q = qkv[:, 0:OUT]
    k = qkv[:, OUT:2 * OUT]
    v = qkv[:, 2 * OUT:3 * OUT]

    lane = jax.lax.broadcasted_iota(jnp.int32, (1, E_P), 1)
    mask_row = jnp.where(lane < E, 0.0, -1e30).astype(f32)
    scale = 1.0 / math.sqrt(D_H)

    heads = []
    for hd in range(HEADS):
        qh = q[:, hd * D_H:(hd + 1) * D_H] * scale
        kh = k[:, hd * D_H:(hd + 1) * D_H]
        vh = v[:, hd * D_H:(hd + 1) * D_H]
        s = jax.lax.dot_general(qh, kh, (((1,), (1,)), ((), ())),
                                preferred_element_type=f32)
        s = s + mask_row
        m = jnp.max(s, axis=1, keepdims=True)
        p = jnp.exp(s - m)
        denom = jnp.sum(p, axis=1, keepdims=True)
        heads.append(jnp.dot(p, vh, preferred_element_type=f32) / denom)

    o = jnp.concatenate(heads, axis=1)
    o = jnp.dot(o, wo_ref[...], preferred_element_type=f32) + bo_ref[...]

    z = jnp.dot(wfct_ref[...], o, preferred_element_type=f32) + bfc_ref[...]
    out_ref[0] = z * 0.5 * (1.0 + jax.lax.erf(z / math.sqrt(2.0)))


@jax.jit
def kernel(node_features, timestamps, edge_features, edge_index,
           W_msg, b_msg, W_qkv, b_qkv, W_o, b_o, W_fc, b_fc):
    f32 = jnp.float32
    node_p = jnp.pad(node_features, ((0, 0), (0, N_P - NUM_NODES), (0, 0)))
    ts_p = jnp.pad(timestamps, ((0, 0), (0, E_P - E)))[:, :, None]
    ef_p = jnp.pad(edge_features, ((0, 0), (0, E_P - E), (0, 0)))
    src = jnp.pad(edge_index[0], (0, E_P - E))[:, None]
    dst = jnp.pad(edge_index[1], (0, E_P - E))[:, None]

    w12 = jnp.pad(
        jnp.concatenate([W_msg[0:2, :], W_msg[2:4, :]], axis=1),
        ((0, 6), (0, 0)))
    w34 = jnp.pad(W_msg[4:14, :], ((0, 6), (0, 0)))
    wfct = jnp.pad(W_fc.T, ((0, N_P - NUM_NODES), (0, E_P - E)))
    bfc = jnp.pad(b_fc, (0, N_P - NUM_NODES))[:, None]
    half = TIME_DIM // 2
    freqs = (1.0 / (10000.0 ** (jnp.arange(half, dtype=f32) / half)))[None, :]

    grid = (B,)
    z = pl.pallas_call(
        _fused_kernel,
        grid=grid,
        in_specs=[
            pl.BlockSpec((1, N_P, NODE_DIM), lambda b: (b, 0, 0)),
            pl.BlockSpec((1, E_P, 1), lambda b: (b, 0, 0)),
            pl.BlockSpec((1, E_P, EDGE_DIM), lambda b: (b, 0, 0)),
            pl.BlockSpec((E_P, 1), lambda b: (0, 0)),
            pl.BlockSpec((E_P, 1), lambda b: (0, 0)),
            pl.BlockSpec((8, 2 * OUT), lambda b: (0, 0)),
            pl.BlockSpec((16, OUT), lambda b: (0, 0)),
            pl.BlockSpec((1, OUT), lambda b: (0, 0)),
            pl.BlockSpec((OUT, 3 * OUT), lambda b: (0, 0)),
            pl.BlockSpec((1, 3 * OUT), lambda b: (0, 0)),
            pl.BlockSpec((OUT, OUT), lambda b: (0, 0)),
            pl.BlockSpec((1, OUT), lambda b: (0, 0)),
            pl.BlockSpec((N_P, E_P), lambda b: (0, 0)),
            pl.BlockSpec((N_P, 1), lambda b: (0, 0)),
            pl.BlockSpec((1, half), lambda b: (0, 0)),
        ],
        out_specs=pl.BlockSpec((1, N_P, OUT), lambda b: (b, 0, 0)),
        out_shape=jax.ShapeDtypeStruct((B, N_P, OUT), f32),
    )(node_p, ts_p, ef_p, src, dst, w12, w34, b_msg[None, :], W_qkv,
      b_qkv[None, :], W_o, b_o[None, :], wfct, bfc, freqs)
    return z[:, :NUM_NODES, :]


# speedup vs baseline: 1.1416x; 1.1098x over previous
import math

import jax
import jax.numpy as jnp
from jax.experimental import pallas as pl
from jax.experimental.pallas import tpu as pltpu

B = 64
NUM_NODES = 325
E = 940
NODE_DIM = 2
EDGE_DIM = 2
TIME_DIM = 8
OUT = 64
HEADS = 2
D_H = OUT // HEADS

N_P = 384
E_P = 1024
E_Q = 944


def _fused_kernel(node_ref, ts_ref, ef_ref, src_ref, dst_ref,
                  w12_ref, w34_ref, bmsg_ref, wqkv_ref, bqkv_ref,
                  wo_ref, bo_ref, wfct_ref, bfc_ref, freqs_ref,
                  out_ref, oh_src_ref, oh_dst_ref):
    f32 = jnp.float32
    b = pl.program_id(0)

    @pl.when(b == 0)
    def _build_onehots():
        n_iota = jax.lax.broadcasted_iota(jnp.int32, (E_P, N_P), 1)
        oh_src_ref[...] = (n_iota == src_ref[...]).astype(f32)
        oh_dst_ref[...] = (n_iota == dst_ref[...]).astype(f32)

    node = node_ref[0]
    p12 = jnp.dot(node, w12_ref[0:2, :], preferred_element_type=f32)
    h = jnp.dot(oh_src_ref[...], p12[:, :OUT], preferred_element_type=f32)
    h = h + jnp.dot(oh_dst_ref[...], p12[:, OUT:], preferred_element_type=f32)

    t = ts_ref[0]
    ang = t * freqs_ref[...]
    edte = jnp.concatenate(
        [ef_ref[0], jnp.sin(ang), jnp.cos(ang),
         jnp.zeros((E_P, 6), dtype=f32)], axis=1)
    h = h + jnp.dot(edte, w34_ref[...], preferred_element_type=f32)
    h = h + bmsg_ref[...]

    qkv = jnp.dot(h, wqkv_ref[...], preferred_element_type=f32) + bqkv_ref[...]
    q = qkv[0:E_Q, 0:OUT]
    k = qkv[:, OUT:2 * OUT]
    v = qkv[:, 2 * OUT:3 * OUT]

    lane = jax.lax.broadcasted_iota(jnp.int32, (1, E_P), 1)
    mask_row = jnp.where(lane < E, 0.0, -1e30).astype(f32)
    scale = 1.0 / math.sqrt(D_H)
    ones_col = jnp.ones((E_P, 1), dtype=f32)

    heads = []
    for hd in range(HEADS):
        qh = q[:, hd * D_H:(hd + 1) * D_H] * scale
        kh = k[:, hd * D_H:(hd + 1) * D_H]
        vh = jnp.concatenate([v[:, hd * D_H:(hd + 1) * D_H], ones_col], axis=1)
        s = jax.lax.dot_general(qh, kh, (((1,), (1,)), ((), ())),
                                preferred_element_type=f32)
        p = jnp.exp(s + mask_row)
        r = jnp.dot(p, vh, preferred_element_type=f32)
        heads.append(r[:, :D_H] / r[:, D_H:D_H + 1])

    o = jnp.concatenate(heads, axis=1)
    o = jnp.dot(o, wo_ref[...], preferred_element_type=f32) + bo_ref[...]

    z = jnp.dot(wfct_ref[...], o, preferred_element_type=f32) + bfc_ref[...]
    out_ref[0] = z * 0.5 * (1.0 + jax.lax.erf(z / math.sqrt(2.0)))


@jax.jit
def kernel(node_features, timestamps, edge_features, edge_index,
           W_msg, b_msg, W_qkv, b_qkv, W_o, b_o, W_fc, b_fc):
    f32 = jnp.float32
    node_p = jnp.pad(node_features, ((0, 0), (0, N_P - NUM_NODES), (0, 0)))
    ts_p = jnp.pad(timestamps, ((0, 0), (0, E_P - E)))[:, :, None]
    ef_p = jnp.pad(edge_features, ((0, 0), (0, E_P - E), (0, 0)))
    src = jnp.pad(edge_index[0], (0, E_P - E))[:, None]
    dst = jnp.pad(edge_index[1], (0, E_P - E))[:, None]

    w12 = jnp.pad(
        jnp.concatenate([W_msg[0:2, :], W_msg[2:4, :]], axis=1),
        ((0, 6), (0, 0)))
    w34 = jnp.pad(W_msg[4:14, :], ((0, 6), (0, 0)))
    wfct = jnp.pad(W_fc.T, ((0, N_P - NUM_NODES), (0, E_Q - E)))
    bfc = jnp.pad(b_fc, (0, N_P - NUM_NODES))[:, None]
    half = TIME_DIM // 2
    freqs = (1.0 / (10000.0 ** (jnp.arange(half, dtype=f32) / half)))[None, :]

    grid = (B,)
    z = pl.pallas_call(
        _fused_kernel,
        grid=grid,
        in_specs=[
            pl.BlockSpec((1, N_P, NODE_DIM), lambda b: (b, 0, 0)),
            pl.BlockSpec((1, E_P, 1), lambda b: (b, 0, 0)),
            pl.BlockSpec((1, E_P, EDGE_DIM), lambda b: (b, 0, 0)),
            pl.BlockSpec((E_P, 1), lambda b: (0, 0)),
            pl.BlockSpec((E_P, 1), lambda b: (0, 0)),
            pl.BlockSpec((8, 2 * OUT), lambda b: (0, 0)),
            pl.BlockSpec((16, OUT), lambda b: (0, 0)),
            pl.BlockSpec((1, OUT), lambda b: (0, 0)),
            pl.BlockSpec((OUT, 3 * OUT), lambda b: (0, 0)),
            pl.BlockSpec((1, 3 * OUT), lambda b: (0, 0)),
            pl.BlockSpec((OUT, OUT), lambda b: (0, 0)),
            pl.BlockSpec((1, OUT), lambda b: (0, 0)),
            pl.BlockSpec((N_P, E_Q), lambda b: (0, 0)),
            pl.BlockSpec((N_P, 1), lambda b: (0, 0)),
            pl.BlockSpec((1, half), lambda b: (0, 0)),
        ],
        out_specs=pl.BlockSpec((1, N_P, OUT), lambda b: (b, 0, 0)),
        out_shape=jax.ShapeDtypeStruct((B, N_P, OUT), f32),
        scratch_shapes=[
            pltpu.VMEM((E_P, N_P), f32),
            pltpu.VMEM((E_P, N_P), f32),
        ],
    )(node_p, ts_p, ef_p, src, dst, w12, w34, b_msg[None, :], W_qkv,
      b_qkv[None, :], W_o, b_o[None, :], wfct, bfc, freqs)
    return z[:, :NUM_NODES, :]


# speedup vs baseline: 1.5906x; 1.3933x over previous
import math

import jax
import jax.numpy as jnp
from jax.experimental import pallas as pl
from jax.experimental.pallas import tpu as pltpu

B = 64
NUM_NODES = 325
E = 940
NODE_DIM = 2
EDGE_DIM = 2
TIME_DIM = 8
OUT = 64
HEADS = 2
D_H = OUT // HEADS

N_P = 384
E_P = 1024
E_Q = 944


def _fused_kernel(node_ref, ts_ref, ef_ref, src_ref, dst_ref,
                  w12_ref, wef_ref, wsc_ref, bmsg_ref, wqkv_ref, bqkv_ref,
                  wo_ref, bo_ref, wfct_ref, bfc_ref, freqs_ref,
                  out_ref, oh_src_ref, oh_dst_ref):
    f32 = jnp.float32
    b = pl.program_id(0)

    @pl.when(b == 0)
    def _build_onehots():
        n_iota = jax.lax.broadcasted_iota(jnp.int32, (E_P, N_P), 1)
        oh_src_ref[...] = (n_iota == src_ref[...]).astype(f32)
        oh_dst_ref[...] = (n_iota == dst_ref[...]).astype(f32)

    node = node_ref[0]
    p12 = jnp.dot(node, w12_ref[0:2, :], preferred_element_type=f32)
    h = jnp.dot(oh_src_ref[...], p12[:, :OUT], preferred_element_type=f32)
    h = h + jnp.dot(oh_dst_ref[...], p12[:, OUT:], preferred_element_type=f32)

    t_row = ts_ref[0]
    ang = freqs_ref[...] * t_row
    sc = jnp.concatenate([jnp.sin(ang), jnp.cos(ang)], axis=0)
    h = h + jax.lax.dot_general(ef_ref[0], wef_ref[...],
                                (((0,), (0,)), ((), ())),
                                preferred_element_type=f32)
    h = h + jax.lax.dot_general(sc, wsc_ref[...],
                                (((0,), (0,)), ((), ())),
                                preferred_element_type=f32)
    h = h + bmsg_ref[...]

    qkv = jnp.dot(h, wqkv_ref[...], preferred_element_type=f32) + bqkv_ref[...]
    q = qkv[0:E_Q, 0:OUT]
    k = qkv[:, OUT:2 * OUT]
    v = qkv[:, 2 * OUT:3 * OUT]

    lane = jax.lax.broadcasted_iota(jnp.int32, (1, E_P), 1)
    mask_row = jnp.where(lane < E, 0.0, -1e30).astype(f32)
    ones_col = jnp.ones((E_P, 1), dtype=f32)

    heads = []
    for hd in range(HEADS):
        qh = q[:, hd * D_H:(hd + 1) * D_H]
        kh = k[:, hd * D_H:(hd + 1) * D_H]
        vh = jnp.concatenate([v[:, hd * D_H:(hd + 1) * D_H], ones_col], axis=1)
        s = jax.lax.dot_general(qh, kh, (((1,), (1,)), ((), ())),
                                preferred_element_type=f32)
        p = jnp.exp(s + mask_row)
        r = jnp.dot(p, vh, preferred_element_type=f32)
        heads.append(r[:, :D_H] * (1.0 / r[:, D_H:D_H + 1]))

    o = jnp.concatenate(heads, axis=1)
    o = jnp.dot(o, wo_ref[...], preferred_element_type=f32) + bo_ref[...]

    z = jnp.dot(wfct_ref[...], o, preferred_element_type=f32) + bfc_ref[...]
    out_ref[0] = z * 0.5 * (1.0 + jax.lax.erf(z / math.sqrt(2.0)))


@jax.jit
def kernel(node_features, timestamps, edge_features, edge_index,
           W_msg, b_msg, W_qkv, b_qkv, W_o, b_o, W_fc, b_fc):
    f32 = jnp.float32
    node_p = jnp.pad(node_features, ((0, 0), (0, N_P - NUM_NODES), (0, 0)))
    ts_p = jnp.pad(timestamps, ((0, 0), (0, E_P - E)))[:, None, :]
    ef_t = jnp.pad(edge_features.transpose(0, 2, 1),
                   ((0, 0), (0, 6), (0, E_P - E)))
    src = jnp.pad(edge_index[0], (0, E_P - E))[:, None]
    dst = jnp.pad(edge_index[1], (0, E_P - E))[:, None]

    w12 = jnp.pad(
        jnp.concatenate([W_msg[0:2, :], W_msg[2:4, :]], axis=1),
        ((0, 6), (0, 0)))
    wef = jnp.pad(W_msg[4:6, :], ((0, 6), (0, 0)))
    wsc = jnp.concatenate(
        [jnp.pad(W_msg[6:10, :], ((0, 4), (0, 0))),
         jnp.pad(W_msg[10:14, :], ((0, 4), (0, 0)))], axis=0)
    scale = 1.0 / math.sqrt(D_H)
    wqkv = jnp.concatenate([W_qkv[:, :OUT] * scale, W_qkv[:, OUT:]], axis=1)
    bqkv = jnp.concatenate([b_qkv[:OUT] * scale, b_qkv[OUT:]])
    wfct = jnp.pad(W_fc.T, ((0, N_P - NUM_NODES), (0, E_Q - E)))
    bfc = jnp.pad(b_fc, (0, N_P - NUM_NODES))[:, None]
    half = TIME_DIM // 2
    freqs = jnp.pad(
        1.0 / (10000.0 ** (jnp.arange(half, dtype=f32) / half)),
        (0, 4))[:, None]

    grid = (B,)
    z = pl.pallas_call(
        _fused_kernel,
        grid=grid,
        in_specs=[
            pl.BlockSpec((1, N_P, NODE_DIM), lambda b: (b, 0, 0)),
            pl.BlockSpec((1, 1, E_P), lambda b: (b, 0, 0)),
            pl.BlockSpec((1, 8, E_P), lambda b: (b, 0, 0)),
            pl.BlockSpec((E_P, 1), lambda b: (0, 0)),
            pl.BlockSpec((E_P, 1), lambda b: (0, 0)),
            pl.BlockSpec((8, 2 * OUT), lambda b: (0, 0)),
            pl.BlockSpec((8, OUT), lambda b: (0, 0)),
            pl.BlockSpec((16, OUT), lambda b: (0, 0)),
            pl.BlockSpec((1, OUT), lambda b: (0, 0)),
            pl.BlockSpec((OUT, 3 * OUT), lambda b: (0, 0)),
            pl.BlockSpec((1, 3 * OUT), lambda b: (0, 0)),
            pl.BlockSpec((OUT, OUT), lambda b: (0, 0)),
            pl.BlockSpec((1, OUT), lambda b: (0, 0)),
            pl.BlockSpec((N_P, E_Q), lambda b: (0, 0)),
            pl.BlockSpec((N_P, 1), lambda b: (0, 0)),
            pl.BlockSpec((8, 1), lambda b: (0, 0)),
        ],
        out_specs=pl.BlockSpec((1, N_P, OUT), lambda b: (b, 0, 0)),
        out_shape=jax.ShapeDtypeStruct((B, N_P, OUT), f32),
        scratch_shapes=[
            pltpu.VMEM((E_P, N_P), f32),
            pltpu.VMEM((E_P, N_P), f32),
        ],
    )(node_p, ts_p, ef_t, src, dst, w12, wef, wsc, b_msg[None, :], wqkv,
      bqkv[None, :], W_o, b_o[None, :], wfct, bfc, freqs)
    return z[:, :NUM_NODES, :]
